# Initial kernel scaffold; baseline (speedup 1.0000x reference)
import functools

import jax
import jax.numpy as jnp
from jax import lax
from jax.experimental import pallas as pl
from jax.experimental.pallas import tpu as pltpu
from jax.experimental.pallas import tpu_sc as plsc

N = 10000
E = 320000
NC = 2
NS = 16
NW = NC * NS
EPT = E // NW
CH = 128
FULL = EPT // CH
TAIL = EPT - FULL * CH
RPT = N // NS
ZR = 125


def _mesh():
    return plsc.VectorSubcoreMesh(core_axis_name="c", subcore_axis_name="s")


def _deg_body(dst_hbm, dpart_hbm, idx_v, idx_t, dcnt_v):
    cid = lax.axis_index("c")
    sid = lax.axis_index("s")
    wid = cid * NS + sid
    ebase = wid * EPT

    def zero(i, _):
        dcnt_v[pl.ds(i * 16, 16)] = jnp.zeros((16,), jnp.float32)
        return _

    lax.fori_loop(0, N // 16, zero, 0)

    ones = jnp.ones((16,), jnp.float32)

    def chunk(i, _):
        pltpu.sync_copy(dst_hbm.at[pl.ds(ebase + i * CH, CH)], idx_v)
        for j in range(CH // 16):
            vec = idx_v[pl.ds(j * 16, 16)]
            plsc.addupdate_scatter(dcnt_v, [vec], ones)
        return _

    lax.fori_loop(0, FULL, chunk, 0)
    pltpu.sync_copy(dst_hbm.at[pl.ds(ebase + FULL * CH, TAIL)], idx_t)
    vec = idx_t[pl.ds(0, 16)]
    plsc.addupdate_scatter(dcnt_v, [vec], ones)
    pltpu.sync_copy(dcnt_v, dpart_hbm.at[wid])


@functools.partial(
    pl.kernel,
    out_type=jax.ShapeDtypeStruct((NW, N), jnp.float32),
    mesh=_mesh(),
    scratch_types=[
        pltpu.VMEM((CH,), jnp.int32),
        pltpu.VMEM((TAIL,), jnp.int32),
        pltpu.VMEM((N,), jnp.float32),
    ],
)
def _deg_kernel(dst_hbm, dpart_hbm, idx_v, idx_t, dcnt_v):
    _deg_body(dst_hbm, dpart_hbm, idx_v, idx_t, dcnt_v)


def _agg_body(F, y_hbm, src_hbm, dst_hbm, zeros_hbm, zp_hbm,
              isrc_v, idst_v, isrc_t, idst_t, buf_v, buf_t, zbuf_v, acc_sh, sem):
    cid = lax.axis_index("c")
    sid = lax.axis_index("s")
    wid = cid * NS + sid
    ebase = wid * EPT
    row0 = sid * RPT

    pltpu.sync_copy(zeros_hbm, zbuf_v)
    for k in range(RPT // ZR):
        pltpu.sync_copy(zbuf_v, acc_sh.at[pl.ds(row0 + k * ZR, ZR)])
    plsc.subcore_barrier()

    def chunk(i, _):
        base = ebase + i * CH
        pltpu.sync_copy(src_hbm.at[pl.ds(base, CH)], isrc_v)
        pltpu.sync_copy(dst_hbm.at[pl.ds(base, CH)], idst_v)
        pltpu.async_copy(y_hbm.at[isrc_v], buf_v, sem).wait()
        pltpu.sync_copy(buf_v, acc_sh.at[idst_v], add=True)
        return _

    lax.fori_loop(0, FULL, chunk, 0)
    base = ebase + FULL * CH
    pltpu.sync_copy(src_hbm.at[pl.ds(base, TAIL)], isrc_t)
    pltpu.sync_copy(dst_hbm.at[pl.ds(base, TAIL)], idst_t)
    pltpu.async_copy(y_hbm.at[isrc_t], buf_t, sem).wait()
    pltpu.sync_copy(buf_t, acc_sh.at[idst_t], add=True)

    plsc.subcore_barrier()
    for k in range(RPT // ZR):
        r = row0 + k * ZR
        pltpu.sync_copy(acc_sh.at[pl.ds(r, ZR)], zbuf_v)
        pltpu.sync_copy(zbuf_v, zp_hbm.at[cid, pl.ds(r, ZR)])


def _make_agg(F):
    @functools.partial(
        pl.kernel,
        out_type=jax.ShapeDtypeStruct((NC, N, F), jnp.float32),
        mesh=_mesh(),
        scratch_types=[
            pltpu.VMEM((CH,), jnp.int32),
            pltpu.VMEM((CH,), jnp.int32),
            pltpu.VMEM((TAIL,), jnp.int32),
            pltpu.VMEM((TAIL,), jnp.int32),
            pltpu.VMEM((CH, F), jnp.float32),
            pltpu.VMEM((TAIL, F), jnp.float32),
            pltpu.VMEM((ZR, F), jnp.float32),
            pltpu.VMEM_SHARED((N, F), jnp.float32),
            pltpu.SemaphoreType.DMA,
        ],
    )
    def agg(y_hbm, src_hbm, dst_hbm, zeros_hbm, zp_hbm, *rest):
        _agg_body(F, y_hbm, src_hbm, dst_hbm, zeros_hbm, zp_hbm, *rest)

    return agg


_agg128 = _make_agg(128)
_agg64 = _make_agg(64)


def _prep_body(dpart_ref, x_ref, s_ref, y_ref):
    deg = 1.0 + jnp.sum(dpart_ref[...], axis=0)
    s = lax.rsqrt(deg)
    s_ref[...] = s
    y_ref[...] = s[:, None] * x_ref[...]


def _prep(dpart, x):
    return pl.pallas_call(
        _prep_body,
        out_shape=(
            jax.ShapeDtypeStruct((N,), jnp.float32),
            jax.ShapeDtypeStruct((N, 128), jnp.float32),
        ),
    )(dpart, x)


RB = 2500


def _layer_mid_body(zp_ref, y_ref, s_ref, W_ref, b_ref, out_ref):
    z = zp_ref[0] + zp_ref[1] + y_ref[...]
    t = s_ref[...][:, None] * z
    h = jnp.dot(t, W_ref[...], preferred_element_type=jnp.float32) + b_ref[...][None, :]
    out_ref[...] = s_ref[...][:, None] * jnp.maximum(h, 0.0)


def _layer_last_body(zp_ref, y_ref, s_ref, W_ref, b_ref, Wn_ref, out_ref):
    z = zp_ref[0] + zp_ref[1] + y_ref[...]
    t = s_ref[...][:, None] * z
    h = jnp.dot(t, W_ref[...], preferred_element_type=jnp.float32) + b_ref[...][None, :]
    h = jnp.maximum(h, 0.0)
    out_ref[...] = s_ref[...][:, None] * jnp.dot(
        h, Wn_ref[...], preferred_element_type=jnp.float32)


def _layer_specs(fo):
    in_specs = [
        pl.BlockSpec((NC, RB, 128), lambda i: (0, i, 0)),
        pl.BlockSpec((RB, 128), lambda i: (i, 0)),
        pl.BlockSpec((RB,), lambda i: (i,)),
        pl.BlockSpec((128, 128), lambda i: (0, 0)),
        pl.BlockSpec((128,), lambda i: (0,)),
    ]
    out_spec = pl.BlockSpec((RB, fo), lambda i: (i, 0))
    return in_specs, out_spec


def _layer_mid(zp, y, s, W, b):
    in_specs, out_spec = _layer_specs(128)
    return pl.pallas_call(
        _layer_mid_body,
        grid=(N // RB,),
        in_specs=in_specs,
        out_specs=out_spec,
        out_shape=jax.ShapeDtypeStruct((N, 128), jnp.float32),
    )(zp, y, s, W, b)


def _layer_last(zp, y, s, W, b, Wn):
    in_specs, out_spec = _layer_specs(64)
    in_specs.append(pl.BlockSpec((128, 64), lambda i: (0, 0)))
    return pl.pallas_call(
        _layer_last_body,
        grid=(N // RB,),
        in_specs=in_specs,
        out_specs=out_spec,
        out_shape=jax.ShapeDtypeStruct((N, 64), jnp.float32),
    )(zp, y, s, W, b, Wn)


def _final_body(zp_ref, y_ref, s_ref, b_ref, out_ref):
    z = zp_ref[0] + zp_ref[1] + y_ref[...]
    o = s_ref[...][:, None] * z + b_ref[...][None, :]
    m = jnp.max(o, axis=1, keepdims=True)
    e = jnp.exp(o - m)
    lse = jnp.log(jnp.sum(e, axis=1, keepdims=True)) + m
    out_ref[...] = o - lse


def _final(zp, y, s, b):
    in_specs = [
        pl.BlockSpec((NC, RB, 64), lambda i: (0, i, 0)),
        pl.BlockSpec((RB, 64), lambda i: (i, 0)),
        pl.BlockSpec((RB,), lambda i: (i,)),
        pl.BlockSpec((64,), lambda i: (0,)),
    ]
    return pl.pallas_call(
        _final_body,
        grid=(N // RB,),
        in_specs=in_specs,
        out_specs=pl.BlockSpec((RB, 64), lambda i: (i, 0)),
        out_shape=jax.ShapeDtypeStruct((N, 64), jnp.float32),
    )(zp, y, s, b)


def kernel(x, edge_index, W1, b1, W2, b2, W3, b3):
    src = edge_index[0]
    dst = edge_index[1]
    z128 = jnp.zeros((ZR, 128), jnp.float32)
    z64 = jnp.zeros((ZR, 64), jnp.float32)

    dpart = _deg_kernel(dst)
    s, y1 = _prep(dpart, x)
    zp1 = _agg128(y1, src, dst, z128)
    y2 = _layer_mid(zp1, y1, s, W1, b1)
    zp2 = _agg128(y2, src, dst, z128)
    y3 = _layer_last(zp2, y2, s, W2, b2, W3)
    zp3 = _agg64(y3, src, dst, z64)
    return _final(zp3, y3, s, b3)


# speedup vs baseline: 14.9659x; 14.9659x over previous
import functools

import jax
import jax.numpy as jnp
from jax import lax
from jax.experimental import pallas as pl
from jax.experimental.pallas import tpu as pltpu
from jax.experimental.pallas import tpu_sc as plsc

N = 10000
E = 320000
NC = 2
NS = 16
NW = NC * NS
EPT = E // NW
CH = 128
FULL = EPT // CH
TAIL = EPT - FULL * CH
NP = 10240
RPT = NP // NS
ZR = 128


def _mesh():
    return plsc.VectorSubcoreMesh(core_axis_name="c", subcore_axis_name="s")


def _deg_body(dst_hbm, dpart_hbm, idx_v, idx_t, dcnt_v):
    cid = lax.axis_index("c")
    sid = lax.axis_index("s")
    wid = cid * NS + sid
    ebase = wid * EPT

    def zero(i, _):
        dcnt_v[pl.ds(i * 16, 16)] = jnp.zeros((16,), jnp.float32)
        return _

    lax.fori_loop(0, N // 16, zero, 0)

    ones = jnp.ones((16,), jnp.float32)

    def chunk(i, _):
        pltpu.sync_copy(dst_hbm.at[pl.ds(ebase + i * CH, CH)], idx_v)
        for j in range(CH // 16):
            vec = idx_v[pl.ds(j * 16, 16)]
            plsc.addupdate_scatter(dcnt_v, [vec], ones)
        return _

    lax.fori_loop(0, FULL, chunk, 0)
    pltpu.sync_copy(dst_hbm.at[pl.ds(ebase + FULL * CH, TAIL)], idx_t)
    vec = idx_t[pl.ds(0, 16)]
    plsc.addupdate_scatter(dcnt_v, [vec], ones)
    pltpu.sync_copy(dcnt_v, dpart_hbm.at[pl.ds(wid * N, N)])


@functools.partial(
    pl.kernel,
    out_type=jax.ShapeDtypeStruct((NW * N,), jnp.float32),
    mesh=_mesh(),
    compiler_params=pltpu.CompilerParams(needs_layout_passes=False),
    scratch_types=[
        pltpu.VMEM((CH,), jnp.int32),
        pltpu.VMEM((TAIL,), jnp.int32),
        pltpu.VMEM((N,), jnp.float32),
    ],
)
def _deg_kernel(dst_hbm, dpart_hbm, idx_v, idx_t, dcnt_v):
    _deg_body(dst_hbm, dpart_hbm, idx_v, idx_t, dcnt_v)


def _agg_body(F, y_hbm, src_hbm, dst_hbm, zeros_hbm, zp_hbm,
              isrc_v, idst_v, isrc_t, idst_t, buf_v, buf_t, zbuf_v, acc_sh, sem):
    cid = lax.axis_index("c")
    sid = lax.axis_index("s")
    wid = cid * NS + sid
    ebase = wid * EPT
    row0 = sid * RPT

    pltpu.sync_copy(zeros_hbm, zbuf_v)
    for k in range(RPT // ZR):
        pltpu.sync_copy(zbuf_v, acc_sh.at[pl.ds(row0 + k * ZR, ZR)])
    plsc.subcore_barrier()

    def chunk(i, _):
        base = ebase + i * CH
        pltpu.sync_copy(src_hbm.at[pl.ds(base, CH)], isrc_v)
        pltpu.sync_copy(dst_hbm.at[pl.ds(base, CH)], idst_v)
        pltpu.async_copy(y_hbm.at[isrc_v], buf_v, sem).wait()
        pltpu.sync_copy(buf_v, acc_sh.at[idst_v], add=True)
        return _

    lax.fori_loop(0, FULL, chunk, 0)
    base = ebase + FULL * CH
    pltpu.sync_copy(src_hbm.at[pl.ds(base, TAIL)], isrc_t)
    pltpu.sync_copy(dst_hbm.at[pl.ds(base, TAIL)], idst_t)
    pltpu.async_copy(y_hbm.at[isrc_t], buf_t, sem).wait()
    pltpu.sync_copy(buf_t, acc_sh.at[idst_t], add=True)

    plsc.subcore_barrier()
    for k in range(RPT // ZR):
        r = row0 + k * ZR
        pltpu.sync_copy(acc_sh.at[pl.ds(r, ZR)], zbuf_v)
        pltpu.sync_copy(zbuf_v, zp_hbm.at[cid, pl.ds(r, ZR)])


def _make_agg(F):
    @functools.partial(
        pl.kernel,
        out_type=jax.ShapeDtypeStruct((NC, NP, F), jnp.float32),
        mesh=_mesh(),
        compiler_params=pltpu.CompilerParams(
            needs_layout_passes=False, use_tc_tiling_on_sc=False),
        scratch_types=[
            pltpu.VMEM((CH,), jnp.int32),
            pltpu.VMEM((CH,), jnp.int32),
            pltpu.VMEM((TAIL,), jnp.int32),
            pltpu.VMEM((TAIL,), jnp.int32),
            pltpu.VMEM((CH, F), jnp.float32),
            pltpu.VMEM((TAIL, F), jnp.float32),
            pltpu.VMEM((ZR, F), jnp.float32),
            pltpu.VMEM_SHARED((NP, F), jnp.float32),
            pltpu.SemaphoreType.DMA,
        ],
    )
    def agg(y_hbm, src_hbm, dst_hbm, zeros_hbm, zp_hbm, *rest):
        _agg_body(F, y_hbm, src_hbm, dst_hbm, zeros_hbm, zp_hbm, *rest)

    return agg


_agg128 = _make_agg(128)
_agg64 = _make_agg(64)


def _prep_body(dpart_ref, x_ref, s_ref, y_ref):
    deg = 1.0 + jnp.sum(dpart_ref[...], axis=0)
    s = lax.rsqrt(deg)[:, None]
    s_ref[...] = s
    y_ref[...] = s * x_ref[...]


def _prep(dpart, x):
    return pl.pallas_call(
        _prep_body,
        out_shape=(
            jax.ShapeDtypeStruct((N, 1), jnp.float32),
            jax.ShapeDtypeStruct((N, 128), jnp.float32),
        ),
    )(dpart, x)


RB = 2000


def _layer_mid_body(zp_ref, y_ref, s_ref, W_ref, b_ref, out_ref):
    z = zp_ref[0] + zp_ref[1] + y_ref[...]
    t = s_ref[...] * z
    h = jnp.dot(t, W_ref[...], preferred_element_type=jnp.float32) + b_ref[...][None, :]
    out_ref[...] = s_ref[...] * jnp.maximum(h, 0.0)


def _layer_last_body(zp_ref, y_ref, s_ref, W_ref, b_ref, Wn_ref, out_ref):
    z = zp_ref[0] + zp_ref[1] + y_ref[...]
    t = s_ref[...] * z
    h = jnp.dot(t, W_ref[...], preferred_element_type=jnp.float32) + b_ref[...][None, :]
    h = jnp.maximum(h, 0.0)
    out_ref[...] = s_ref[...] * jnp.dot(
        h, Wn_ref[...], preferred_element_type=jnp.float32)


def _layer_specs(fo):
    in_specs = [
        pl.BlockSpec((NC, RB, 128), lambda i: (0, i, 0)),
        pl.BlockSpec((RB, 128), lambda i: (i, 0)),
        pl.BlockSpec((RB, 1), lambda i: (i, 0)),
        pl.BlockSpec((128, 128), lambda i: (0, 0)),
        pl.BlockSpec((128,), lambda i: (0,)),
    ]
    out_spec = pl.BlockSpec((RB, fo), lambda i: (i, 0))
    return in_specs, out_spec


def _layer_mid(zp, y, s, W, b):
    in_specs, out_spec = _layer_specs(128)
    return pl.pallas_call(
        _layer_mid_body,
        grid=(N // RB,),
        in_specs=in_specs,
        out_specs=out_spec,
        out_shape=jax.ShapeDtypeStruct((N, 128), jnp.float32),
    )(zp, y, s, W, b)


def _layer_last(zp, y, s, W, b, Wn):
    in_specs, out_spec = _layer_specs(64)
    in_specs.append(pl.BlockSpec((128, 64), lambda i: (0, 0)))
    return pl.pallas_call(
        _layer_last_body,
        grid=(N // RB,),
        in_specs=in_specs,
        out_specs=out_spec,
        out_shape=jax.ShapeDtypeStruct((N, 64), jnp.float32),
    )(zp, y, s, W, b, Wn)


def _final_body(zp_ref, y_ref, s_ref, b_ref, out_ref):
    z = zp_ref[0] + zp_ref[1] + y_ref[...]
    o = s_ref[...] * z + b_ref[...][None, :]
    m = jnp.max(o, axis=1, keepdims=True)
    e = jnp.exp(o - m)
    lse = jnp.log(jnp.sum(e, axis=1, keepdims=True)) + m
    out_ref[...] = o - lse


def _final(zp, y, s, b):
    in_specs = [
        pl.BlockSpec((NC, RB, 64), lambda i: (0, i, 0)),
        pl.BlockSpec((RB, 64), lambda i: (i, 0)),
        pl.BlockSpec((RB, 1), lambda i: (i, 0)),
        pl.BlockSpec((64,), lambda i: (0,)),
    ]
    return pl.pallas_call(
        _final_body,
        grid=(N // RB,),
        in_specs=in_specs,
        out_specs=pl.BlockSpec((RB, 64), lambda i: (i, 0)),
        out_shape=jax.ShapeDtypeStruct((N, 64), jnp.float32),
    )(zp, y, s, b)


def kernel(x, edge_index, W1, b1, W2, b2, W3, b3):
    src = edge_index[0]
    dst = edge_index[1]
    z128 = jnp.zeros((ZR, 128), jnp.float32)
    z64 = jnp.zeros((ZR, 64), jnp.float32)

    dpart = _deg_kernel(dst).reshape(NW, N)
    s, y1 = _prep(dpart, x)
    zp1 = _agg128(y1, src, dst, z128)
    y2 = _layer_mid(zp1, y1, s, W1, b1)
    zp2 = _agg128(y2, src, dst, z128)
    y3 = _layer_last(zp2, y2, s, W2, b2, W3)
    zp3 = _agg64(y3, src, dst, z64)
    return _final(zp3, y3, s, b3)


# speedup vs baseline: 26.5045x; 1.7710x over previous
import functools

import jax
import jax.numpy as jnp
from jax import lax
from jax.experimental import pallas as pl
from jax.experimental.pallas import tpu as pltpu
from jax.experimental.pallas import tpu_sc as plsc

N = 10000
E = 320000
NC = 2
NS = 16
NW = NC * NS
CH = 64
NCHUNK = E // CH
CPW = NCHUNK // NW
XTR = NCHUNK - NW * CPW
NP = 10240
RPT = NP // NS
ZR = 64


def _mesh():
    return plsc.VectorSubcoreMesh(core_axis_name="c", subcore_axis_name="s")


def _deg_body(dst_hbm, dpart_hbm, dall_v, dx_v, dcnt_v):
    cid = lax.axis_index("c")
    sid = lax.axis_index("s")
    wid = cid * NS + sid

    def zero(i, _):
        dcnt_v[pl.ds(i * 16, 16)] = jnp.zeros((16,), jnp.float32)
        return _

    lax.fori_loop(0, N // 16, zero, 0)

    pltpu.sync_copy(dst_hbm.at[pl.ds(wid * CPW, CPW)], dall_v)
    ones = jnp.ones((16,), jnp.float32)

    def chunk(i, _):
        for j in range(CH // 16):
            vec = dall_v[i, pl.ds(j * 16, 16)]
            plsc.addupdate_scatter(dcnt_v, [vec], ones)
        return _

    lax.fori_loop(0, CPW, chunk, 0)

    @pl.when(wid < XTR)
    def _():
        pltpu.sync_copy(dst_hbm.at[pl.ds(NW * CPW + wid, 1)], dx_v)
        for j in range(CH // 16):
            vec = dx_v[0, pl.ds(j * 16, 16)]
            plsc.addupdate_scatter(dcnt_v, [vec], ones)

    pltpu.sync_copy(dcnt_v, dpart_hbm.at[pl.ds(wid * N, N)])


@functools.partial(
    pl.kernel,
    out_type=jax.ShapeDtypeStruct((NW * N,), jnp.float32),
    mesh=_mesh(),
    compiler_params=pltpu.CompilerParams(
        needs_layout_passes=False, use_tc_tiling_on_sc=False),
    scratch_types=[
        pltpu.VMEM((CPW, CH), jnp.int32),
        pltpu.VMEM((1, CH), jnp.int32),
        pltpu.VMEM((N,), jnp.float32),
    ],
)
def _deg_kernel(dst_hbm, dpart_hbm, dall_v, dx_v, dcnt_v):
    _deg_body(dst_hbm, dpart_hbm, dall_v, dx_v, dcnt_v)


def _agg_body(F, y_hbm, src_hbm, dst_hbm, zeros_hbm, zp_hbm,
              isrc_v, idst_v, isx_v, idxx_v, buf0, buf1, acc_sh,
              g0, g1, s0, s1):
    cid = lax.axis_index("c")
    sid = lax.axis_index("s")
    wid = cid * NS + sid
    row0 = sid * RPT

    pltpu.sync_copy(zeros_hbm, buf0)
    for k in range(RPT // ZR):
        pltpu.sync_copy(buf0, acc_sh.at[pl.ds(row0 + k * ZR, ZR)])

    pltpu.sync_copy(src_hbm.at[pl.ds(wid * CPW, CPW)], isrc_v)
    pltpu.sync_copy(dst_hbm.at[pl.ds(wid * CPW, CPW)], idst_v)
    plsc.subcore_barrier()

    bufs = (buf0, buf1)
    gsem = (g0, g1)
    ssem = (s0, s1)

    def gstart(i, b):
        pltpu.async_copy(y_hbm.at[isrc_v.at[i]], bufs[b], gsem[b])

    def gwait(b):
        pltpu.make_async_copy(y_hbm.at[isrc_v.at[0]], bufs[b], gsem[b]).wait()

    def sstart(i, b):
        pltpu.async_copy(bufs[b], acc_sh.at[idst_v.at[i]], ssem[b], add=True)

    def swait(b):
        pltpu.make_async_copy(bufs[b], acc_sh.at[idst_v.at[0]], ssem[b]).wait()

    gstart(0, 0)
    gstart(1, 1)
    gwait(0)
    sstart(0, 0)

    def pair(t, _):
        i = 1 + 2 * t
        swait(0)
        gstart(i + 1, 0)
        gwait(1)
        sstart(i, 1)
        swait(1)
        gstart(i + 2, 1)
        gwait(0)
        sstart(i + 1, 0)
        return _

    lax.fori_loop(0, (CPW - 2) // 2, pair, 0)
    swait(0)
    gwait(1)
    sstart(CPW - 1, 1)
    swait(1)

    @pl.when(wid < XTR)
    def _():
        pltpu.sync_copy(src_hbm.at[pl.ds(NW * CPW + wid, 1)], isx_v)
        pltpu.sync_copy(dst_hbm.at[pl.ds(NW * CPW + wid, 1)], idxx_v)
        pltpu.async_copy(y_hbm.at[isx_v.at[0]], buf0, g0).wait()
        pltpu.sync_copy(buf0, acc_sh.at[idxx_v.at[0]], add=True)

    plsc.subcore_barrier()
    for k in range(RPT // ZR):
        r = row0 + k * ZR
        pltpu.sync_copy(acc_sh.at[pl.ds(r, ZR)], buf0)
        pltpu.sync_copy(buf0, zp_hbm.at[cid, pl.ds(r, ZR)])


def _make_agg(F):
    @functools.partial(
        pl.kernel,
        out_type=jax.ShapeDtypeStruct((NC, NP, F), jnp.float32),
        mesh=_mesh(),
        compiler_params=pltpu.CompilerParams(
            needs_layout_passes=False, use_tc_tiling_on_sc=False),
        scratch_types=[
            pltpu.VMEM((CPW, CH), jnp.int32),
            pltpu.VMEM((CPW, CH), jnp.int32),
            pltpu.VMEM((1, CH), jnp.int32),
            pltpu.VMEM((1, CH), jnp.int32),
            pltpu.VMEM((CH, F), jnp.float32),
            pltpu.VMEM((CH, F), jnp.float32),
            pltpu.VMEM_SHARED((NP, F), jnp.float32),
            pltpu.SemaphoreType.DMA,
            pltpu.SemaphoreType.DMA,
            pltpu.SemaphoreType.DMA,
            pltpu.SemaphoreType.DMA,
        ],
    )
    def agg(y_hbm, src_hbm, dst_hbm, zeros_hbm, zp_hbm, *rest):
        _agg_body(F, y_hbm, src_hbm, dst_hbm, zeros_hbm, zp_hbm, *rest)

    return agg


_agg128 = _make_agg(128)
_agg64 = _make_agg(64)


def _prep_body(dpart_ref, x_ref, s_ref, y_ref):
    deg = 1.0 + jnp.sum(dpart_ref[...], axis=0)
    s = lax.rsqrt(deg)[:, None]
    s_ref[...] = s
    y_ref[...] = s * x_ref[...]


def _prep(dpart, x):
    return pl.pallas_call(
        _prep_body,
        out_shape=(
            jax.ShapeDtypeStruct((N, 1), jnp.float32),
            jax.ShapeDtypeStruct((N, 128), jnp.float32),
        ),
    )(dpart, x)


RB = 2000


def _layer_mid_body(zp_ref, y_ref, s_ref, W_ref, b_ref, out_ref):
    z = zp_ref[0] + zp_ref[1] + y_ref[...]
    t = s_ref[...] * z
    h = jnp.dot(t, W_ref[...], preferred_element_type=jnp.float32) + b_ref[...][None, :]
    out_ref[...] = s_ref[...] * jnp.maximum(h, 0.0)


def _layer_last_body(zp_ref, y_ref, s_ref, W_ref, b_ref, Wn_ref, out_ref):
    z = zp_ref[0] + zp_ref[1] + y_ref[...]
    t = s_ref[...] * z
    h = jnp.dot(t, W_ref[...], preferred_element_type=jnp.float32) + b_ref[...][None, :]
    h = jnp.maximum(h, 0.0)
    out_ref[...] = s_ref[...] * jnp.dot(
        h, Wn_ref[...], preferred_element_type=jnp.float32)


def _layer_specs(fo):
    in_specs = [
        pl.BlockSpec((NC, RB, 128), lambda i: (0, i, 0)),
        pl.BlockSpec((RB, 128), lambda i: (i, 0)),
        pl.BlockSpec((RB, 1), lambda i: (i, 0)),
        pl.BlockSpec((128, 128), lambda i: (0, 0)),
        pl.BlockSpec((128,), lambda i: (0,)),
    ]
    out_spec = pl.BlockSpec((RB, fo), lambda i: (i, 0))
    return in_specs, out_spec


def _layer_mid(zp, y, s, W, b):
    in_specs, out_spec = _layer_specs(128)
    return pl.pallas_call(
        _layer_mid_body,
        grid=(N // RB,),
        in_specs=in_specs,
        out_specs=out_spec,
        out_shape=jax.ShapeDtypeStruct((N, 128), jnp.float32),
    )(zp, y, s, W, b)


def _layer_last(zp, y, s, W, b, Wn):
    in_specs, out_spec = _layer_specs(64)
    in_specs.append(pl.BlockSpec((128, 64), lambda i: (0, 0)))
    return pl.pallas_call(
        _layer_last_body,
        grid=(N // RB,),
        in_specs=in_specs,
        out_specs=out_spec,
        out_shape=jax.ShapeDtypeStruct((N, 64), jnp.float32),
    )(zp, y, s, W, b, Wn)


def _final_body(zp_ref, y_ref, s_ref, b_ref, out_ref):
    z = zp_ref[0] + zp_ref[1] + y_ref[...]
    o = s_ref[...] * z + b_ref[...][None, :]
    m = jnp.max(o, axis=1, keepdims=True)
    e = jnp.exp(o - m)
    lse = jnp.log(jnp.sum(e, axis=1, keepdims=True)) + m
    out_ref[...] = o - lse


def _final(zp, y, s, b):
    in_specs = [
        pl.BlockSpec((NC, RB, 64), lambda i: (0, i, 0)),
        pl.BlockSpec((RB, 64), lambda i: (i, 0)),
        pl.BlockSpec((RB, 1), lambda i: (i, 0)),
        pl.BlockSpec((64,), lambda i: (0,)),
    ]
    return pl.pallas_call(
        _final_body,
        grid=(N // RB,),
        in_specs=in_specs,
        out_specs=pl.BlockSpec((RB, 64), lambda i: (i, 0)),
        out_shape=jax.ShapeDtypeStruct((N, 64), jnp.float32),
    )(zp, y, s, b)


def kernel(x, edge_index, W1, b1, W2, b2, W3, b3):
    src = edge_index[0].reshape(NCHUNK, CH)
    dst = edge_index[1].reshape(NCHUNK, CH)
    z128 = jnp.zeros((ZR, 128), jnp.float32)
    z64 = jnp.zeros((ZR, 64), jnp.float32)

    dpart = _deg_kernel(dst).reshape(NW, N)
    s, y1 = _prep(dpart, x)
    zp1 = _agg128(y1, src, dst, z128)
    y2 = _layer_mid(zp1, y1, s, W1, b1)
    zp2 = _agg128(y2, src, dst, z128)
    y3 = _layer_last(zp2, y2, s, W2, b2, W3)
    zp3 = _agg64(y3, src, dst, z64)
    return _final(zp3, y3, s, b3)
